# Initial kernel scaffold; baseline (speedup 1.0000x reference)
import jax
import jax.numpy as jnp
from jax.experimental import pallas as pl


def kernel(extended_coord, extended_atype, nlist, tab_info, tab_data):
    raise NotImplementedError("write your pallas kernel here")


# speedup vs baseline: 70.8497x; 70.8497x over previous
import functools

import jax
import jax.numpy as jnp
from jax import lax
from jax.experimental import pallas as pl
from jax.experimental.pallas import tpu as pltpu
from jax.experimental.pallas import tpu_sc as plsc

_NC = 2
_NS = 16
_NW = _NC * _NS
_L = 16


def _sc_body(nloc, nnei, ntypes, nspline,
             cx_h, cy_h, cz_h, atype_h, nl_h, t3_h, t2_h, t1_h, t0_h, par_h,
             out_h,
             cx, cy, cz, at, nl, t3, t2, t1, t0, par, ov):
    apw = nloc // _NW
    groups = apw // _L
    wid = lax.axis_index("s") * _NC + lax.axis_index("c")
    base = wid * apw

    pltpu.sync_copy(cx_h, cx)
    pltpu.sync_copy(cy_h, cy)
    pltpu.sync_copy(cz_h, cz)
    pltpu.sync_copy(atype_h, at)
    pltpu.sync_copy(nl_h.at[pl.ds(base * nnei, apw * nnei)], nl)
    pltpu.sync_copy(t3_h, t3)
    pltpu.sync_copy(t2_h, t2)
    pltpu.sync_copy(t1_h, t1)
    pltpu.sync_copy(t0_h, t0)
    pltpu.sync_copy(par_h, par)

    rminv = par[pl.ds(0, _L)]
    hiv = par[pl.ds(_L, _L)]
    lane = lax.iota(jnp.int32, _L)
    oob_uu = jnp.full((_L,), float(nspline + 1), jnp.float32)
    zero16 = jnp.zeros((_L,), jnp.float32)

    for g in range(groups):
        gbase = base + g * _L
        aidx = gbase + lane
        xi = plsc.load_gather(cx, [aidx])
        yi = plsc.load_gather(cy, [aidx])
        zi = plsc.load_gather(cz, [aidx])
        it = plsc.load_gather(at, [aidx])
        itb = it * (ntypes * nspline)
        lane_off = lane * nnei + g * _L * nnei

        def nbody(n, acc):
            jraw = plsc.load_gather(nl, [lane_off + n])
            j = jnp.maximum(jraw, 0)
            xj = plsc.load_gather(cx, [j])
            yj = plsc.load_gather(cy, [j])
            zj = plsc.load_gather(cz, [j])
            jt = plsc.load_gather(at, [j])
            dx = xi - xj
            dy = yi - yj
            dz = zi - zj
            d2 = dx * dx + dy * dy + dz * dz
            seed = jnp.int32(0x5F3759DF) - (plsc.bitcast(d2, jnp.int32) >> 1)
            yk = plsc.bitcast(seed, jnp.float32)
            h = d2 * jnp.float32(0.5)
            yk = yk * (jnp.float32(1.5) - h * yk * yk)
            yk = yk * (jnp.float32(1.5) - h * yk * yk)
            yk = yk * (jnp.float32(1.5) - h * yk * yk)
            rr = d2 * yk
            uu = (rr - rminv) * hiv
            uu = jnp.where(jraw == -1, oob_uu, uu)
            sidx = uu.astype(jnp.int32)
            frac = uu - sidx.astype(jnp.float32)
            cidx = jnp.clip(sidx, 0, nspline - 1)
            flat = itb + jt * nspline + cidx
            a3 = plsc.load_gather(t3, [flat])
            a2 = plsc.load_gather(t2, [flat])
            a1 = plsc.load_gather(t1, [flat])
            a0 = plsc.load_gather(t0, [flat])
            ener = ((a3 * frac + a2) * frac + a1) * frac + a0
            ener = jnp.where(sidx >= nspline, zero16, ener)
            return acc + ener

        acc = lax.fori_loop(0, nnei, nbody, jnp.zeros((_L,), jnp.float32))
        ov[pl.ds(g * _L, _L)] = acc * jnp.float32(0.5)

    pltpu.sync_copy(ov, out_h.at[pl.ds(base, apw)])


def kernel(extended_coord, extended_atype, nlist, tab_info, tab_data):
    nframes, nall, _ = extended_coord.shape
    _, nloc, nnei = nlist.shape
    ntypes = tab_data.shape[0]
    nspline = tab_data.shape[2]

    cx_a = extended_coord[0, :, 0]
    cy_a = extended_coord[0, :, 1]
    cz_a = extended_coord[0, :, 2]
    atype = extended_atype[0]
    nl_flat = nlist[0].reshape(nloc * nnei)
    tabT = tab_data.reshape(ntypes * ntypes * nspline, 4).T
    t3_a, t2_a, t1_a, t0_a = tabT[0], tabT[1], tabT[2], tabT[3]
    rmin = tab_info[0]
    hi = 1.0 / tab_info[1]
    params = jnp.concatenate([
        jnp.full((_L,), rmin, jnp.float32),
        jnp.full((_L,), hi, jnp.float32),
    ])

    apw = nloc // _NW
    tdim = ntypes * ntypes * nspline
    mesh = plsc.VectorSubcoreMesh(core_axis_name="c", subcore_axis_name="s",
                                  num_cores=_NC, num_subcores=_NS)
    body = functools.partial(_sc_body, nloc, nnei, ntypes, nspline)
    out = pl.kernel(
        body,
        out_type=jax.ShapeDtypeStruct((nloc,), jnp.float32),
        mesh=mesh,
        compiler_params=pltpu.CompilerParams(needs_layout_passes=False),
        scratch_types=[
            pltpu.VMEM((nall,), jnp.float32),
            pltpu.VMEM((nall,), jnp.float32),
            pltpu.VMEM((nall,), jnp.float32),
            pltpu.VMEM((nall,), jnp.int32),
            pltpu.VMEM((apw * nnei,), jnp.int32),
            pltpu.VMEM((tdim,), jnp.float32),
            pltpu.VMEM((tdim,), jnp.float32),
            pltpu.VMEM((tdim,), jnp.float32),
            pltpu.VMEM((tdim,), jnp.float32),
            pltpu.VMEM((2 * _L,), jnp.float32),
            pltpu.VMEM((apw,), jnp.float32),
        ],
    )(cx_a, cy_a, cz_a, atype, nl_flat, t3_a, t2_a, t1_a, t0_a, params)
    return out.reshape(nframes, nloc)
